# Initial kernel scaffold; baseline (speedup 1.0000x reference)
import jax
import jax.numpy as jnp
from jax.experimental import pallas as pl


def kernel(idx, depth):
    raise NotImplementedError("write your pallas kernel here")


# speedup vs baseline: 140.6459x; 140.6459x over previous
import functools

import jax
import jax.numpy as jnp
from jax import lax
from jax.experimental import pallas as pl
from jax.experimental.pallas import tpu as pltpu
from jax.experimental.pallas import tpu_sc as plsc

VOCAB = 1000
TBL_PAD = 1024
NC = 2
NS = 16
NW = NC * NS
LANES = 16


def _gather_clip_body(n_per_w, table_hbm, idx_hbm, out_hbm,
                      table_v, idx_v, out_v):
    wid = lax.axis_index("s") * NC + lax.axis_index("c")
    base = wid * n_per_w

    pltpu.sync_copy(table_hbm, table_v)
    pltpu.sync_copy(idx_hbm.at[pl.ds(base, n_per_w)], idx_v)

    def clip_body(j, _):
        t = table_v[pl.ds(j * LANES, LANES)]
        table_v[pl.ds(j * LANES, LANES)] = jnp.minimum(
            jnp.maximum(t, -1.0), 1.0)
        return _

    lax.fori_loop(0, TBL_PAD // LANES, clip_body, None)

    def gather_body(i, _):
        iv = idx_v[pl.ds(i * LANES, LANES)]
        out_v[pl.ds(i * LANES, LANES)] = plsc.load_gather(table_v, [iv])
        return _

    lax.fori_loop(0, n_per_w // LANES, gather_body, None)

    pltpu.sync_copy(out_v, out_hbm.at[pl.ds(base, n_per_w)])


@functools.partial(jax.jit, static_argnames=("n",))
def _run(idx_flat, table_pad, n):
    n_per_w = n // NW
    mesh = plsc.VectorSubcoreMesh(core_axis_name="c", subcore_axis_name="s")
    body = functools.partial(_gather_clip_body, n_per_w)
    return pl.kernel(
        body,
        out_type=jax.ShapeDtypeStruct((n,), jnp.float32),
        mesh=mesh,
        scratch_types=[
            pltpu.VMEM((TBL_PAD,), jnp.float32),
            pltpu.VMEM((n_per_w,), jnp.int32),
            pltpu.VMEM((n_per_w,), jnp.float32),
        ],
        compiler_params=pltpu.CompilerParams(needs_layout_passes=False),
    )(table_pad, idx_flat)


def kernel(idx, depth):
    b, f = idx.shape
    n = b * f
    idx_flat = idx.reshape((n,))
    table_pad = jnp.pad(depth.reshape((VOCAB,)), (0, TBL_PAD - VOCAB))
    out_flat = _run(idx_flat, table_pad, n)
    return out_flat.reshape((b, f, 1))


# speedup vs baseline: 168.4727x; 1.1979x over previous
import functools

import jax
import jax.numpy as jnp
from jax import lax
from jax.experimental import pallas as pl
from jax.experimental.pallas import tpu as pltpu
from jax.experimental.pallas import tpu_sc as plsc

VOCAB = 1000
TBL_PAD = 1024
NC = 2
NS = 16
NW = NC * NS
LANES = 16


def _gather_clip_body(n_per_w, table_hbm, idx_hbm, out_hbm,
                      table_v, idx_v, out_v):
    wid = lax.axis_index("s") * NC + lax.axis_index("c")
    base = wid * n_per_w

    pltpu.sync_copy(table_hbm, table_v)
    pltpu.sync_copy(idx_hbm.at[pl.ds(base, n_per_w)], idx_v)

    @plsc.parallel_loop(0, TBL_PAD // LANES, unroll=4)
    def clip_body(j):
        t = table_v[pl.ds(j * LANES, LANES)]
        table_v[pl.ds(j * LANES, LANES)] = jnp.minimum(
            jnp.maximum(t, -1.0), 1.0)

    @plsc.parallel_loop(0, n_per_w // LANES, unroll=8)
    def gather_body(i):
        iv = idx_v[pl.ds(i * LANES, LANES)]
        out_v[pl.ds(i * LANES, LANES)] = plsc.load_gather(table_v, [iv])

    pltpu.sync_copy(out_v, out_hbm.at[pl.ds(base, n_per_w)])


@functools.partial(jax.jit, static_argnames=("n",))
def _run(idx_flat, table_pad, n):
    n_per_w = n // NW
    mesh = plsc.VectorSubcoreMesh(core_axis_name="c", subcore_axis_name="s")
    body = functools.partial(_gather_clip_body, n_per_w)
    return pl.kernel(
        body,
        out_type=jax.ShapeDtypeStruct((n,), jnp.float32),
        mesh=mesh,
        scratch_types=[
            pltpu.VMEM((TBL_PAD,), jnp.float32),
            pltpu.VMEM((n_per_w,), jnp.int32),
            pltpu.VMEM((n_per_w,), jnp.float32),
        ],
        compiler_params=pltpu.CompilerParams(needs_layout_passes=False),
    )(table_pad, idx_flat)


def kernel(idx, depth):
    b, f = idx.shape
    n = b * f
    idx_flat = idx.reshape((n,))
    table_pad = jnp.pad(depth.reshape((VOCAB,)), (0, TBL_PAD - VOCAB))
    out_flat = _run(idx_flat, table_pad, n)
    return out_flat.reshape((b, f, 1))


# speedup vs baseline: 264.0088x; 1.5671x over previous
import functools

import jax
import jax.numpy as jnp
from jax import lax
from jax.experimental import pallas as pl
from jax.experimental.pallas import tpu as pltpu
from jax.experimental.pallas import tpu_sc as plsc

VOCAB = 1000
TBL_PAD = 1024
NC = 2
NS = 16
NW = NC * NS
LANES = 16


def _gather_clip_body(rows_per_w, chunk_rows, fields, table_hbm, idx_hbm,
                      out_hbm, table_v, idx_v, out_v):
    wid = lax.axis_index("s") * NC + lax.axis_index("c")
    row0 = wid * rows_per_w

    pltpu.sync_copy(table_hbm, table_v)

    @plsc.parallel_loop(0, TBL_PAD // LANES, unroll=4)
    def clip_body(j):
        t = table_v[pl.ds(j * LANES, LANES)]
        table_v[pl.ds(j * LANES, LANES)] = jnp.minimum(
            jnp.maximum(t, -1.0), 1.0)

    n_full = fields // LANES
    tail = fields - LANES

    for c in range(rows_per_w // chunk_rows):
        base = row0 + c * chunk_rows
        pltpu.sync_copy(idx_hbm.at[pl.ds(base, chunk_rows)], idx_v)

        @plsc.parallel_loop(0, chunk_rows, unroll=4)
        def gather_row(r):
            for w in range(n_full):
                iv = idx_v[r, pl.ds(w * LANES, LANES)]
                out_v[r, pl.ds(w * LANES, LANES)] = plsc.load_gather(
                    table_v, [iv])
            iv = idx_v[r, pl.ds(tail, LANES)]
            out_v[r, pl.ds(tail, LANES)] = plsc.load_gather(table_v, [iv])

        pltpu.sync_copy(out_v, out_hbm.at[pl.ds(base, chunk_rows)])


@functools.partial(jax.jit, static_argnames=("rows", "fields"))
def _run(idx, table_pad, rows, fields):
    rows_per_w = rows // NW
    chunk_rows = rows_per_w // 2
    mesh = plsc.VectorSubcoreMesh(core_axis_name="c", subcore_axis_name="s")
    body = functools.partial(_gather_clip_body, rows_per_w, chunk_rows,
                             fields)
    return pl.kernel(
        body,
        out_type=jax.ShapeDtypeStruct((rows, fields), jnp.float32),
        mesh=mesh,
        scratch_types=[
            pltpu.VMEM((TBL_PAD,), jnp.float32),
            pltpu.VMEM((chunk_rows, fields), jnp.int32),
            pltpu.VMEM((chunk_rows, fields), jnp.float32),
        ],
        compiler_params=pltpu.CompilerParams(needs_layout_passes=False),
    )(table_pad, idx)


def kernel(idx, depth):
    b, f = idx.shape
    table_pad = jnp.pad(depth.reshape((VOCAB,)), (0, TBL_PAD - VOCAB))
    out2d = _run(idx, table_pad, b, f)
    return out2d[..., None]


# speedup vs baseline: 329.4631x; 1.2479x over previous
import functools

import jax
import jax.numpy as jnp
from jax import lax
from jax.experimental import pallas as pl
from jax.experimental.pallas import tpu as pltpu
from jax.experimental.pallas import tpu_sc as plsc

VOCAB = 1000
NC = 2
NS = 16
NW = NC * NS
LANES = 16
SEG = 2048


def _gather_clip_body(batch, fields, table_hbm, idxt_hbm, out_hbm,
                      table_v, idx_v, out_v, sem):
    wid = lax.axis_index("s") * NC + lax.axis_index("c")
    n_per_w = (batch * fields) // NW
    nseg = n_per_w // SEG
    segs_per_row = batch // SEG

    pltpu.sync_copy(table_hbm, table_v)

    def clip_at(off):
        t = table_v[pl.ds(off, LANES)]
        table_v[pl.ds(off, LANES)] = jnp.minimum(jnp.maximum(t, -1.0), 1.0)

    @plsc.parallel_loop(0, VOCAB // LANES, unroll=4)
    def clip_body(j):
        clip_at(j * LANES)

    clip_at(VOCAB - LANES)

    copies = []
    for j in range(nseg):
        m = nseg * wid + j
        f = m // segs_per_row
        b = SEG * lax.rem(m, segs_per_row)
        copies.append(pltpu.async_copy(
            idxt_hbm.at[pl.ds(f, 1), pl.ds(b, SEG)],
            idx_v.at[pl.ds(j, 1)], sem))
    for c in copies:
        c.wait()

    @plsc.parallel_loop(0, nseg)
    def gather_seg(j):
        for k in range(SEG // LANES):
            iv = idx_v[j, pl.ds(k * LANES, LANES)]
            out_v[pl.ds(j * SEG + k * LANES, LANES)] = plsc.load_gather(
                table_v, [iv])

    pltpu.sync_copy(out_v, out_hbm.at[pl.ds(wid * n_per_w, n_per_w)])


@functools.partial(jax.jit, static_argnames=("batch", "fields"))
def _run(idxt, table, batch, fields):
    n_per_w = (batch * fields) // NW
    mesh = plsc.VectorSubcoreMesh(core_axis_name="c", subcore_axis_name="s")
    body = functools.partial(_gather_clip_body, batch, fields)
    return pl.kernel(
        body,
        out_type=jax.ShapeDtypeStruct((batch * fields,), jnp.float32),
        mesh=mesh,
        scratch_types=[
            pltpu.VMEM((VOCAB,), jnp.float32),
            pltpu.VMEM((n_per_w // SEG, SEG), jnp.int32),
            pltpu.VMEM((n_per_w,), jnp.float32),
            pltpu.SemaphoreType.DMA,
        ],
        compiler_params=pltpu.CompilerParams(needs_layout_passes=False),
    )(table, idxt)


def kernel(idx, depth):
    b, f = idx.shape
    flat = _run(idx.T, depth.reshape((VOCAB,)), b, f)
    return jnp.transpose(flat.reshape((f, b, 1)), (1, 0, 2))
